# baseline (speedup 1.0000x reference)
import jax
import jax.numpy as jnp
from jax.experimental import pallas as pl
from jax.experimental.pallas import tpu as pltpu

L = 50
B = 16
V = 100000
E = 32
H = 64
NROWS = (L - 2) * B
VB = 4096
NVB = (V + VB - 1) // VB

NEG_INF = float("-inf")


def _gru_kernel(x_ref, wihT_f_ref, wihT_b_ref, whhT_ref, bih_f_ref,
                bih_b_ref, bhh_ref, out_ref, gif_ref, gib_ref):
    gif_ref[...] = jnp.dot(x_ref[...], wihT_f_ref[...],
                           preferred_element_type=jnp.float32) + bih_f_ref[...]
    gib_ref[...] = jnp.dot(x_ref[...], wihT_b_ref[...],
                           preferred_element_type=jnp.float32) + bih_b_ref[...]
    whhT = whhT_ref[...]
    bhh = bhh_ref[...]

    def body(i, h2):
        gh2 = jnp.dot(h2, whhT, preferred_element_type=jnp.float32) + bhh
        gi = jnp.concatenate([gif_ref[pl.ds(i * B, B), :],
                              gib_ref[pl.ds((L - 1 - i) * B, B), :]], axis=0)
        gh = jnp.concatenate([gh2[0:B, 0:3 * H],
                              gh2[B:2 * B, 3 * H:6 * H]], axis=0)
        r = jax.nn.sigmoid(gi[:, 0:H] + gh[:, 0:H])
        z = jax.nn.sigmoid(gi[:, H:2 * H] + gh[:, H:2 * H])
        n = jnp.tanh(gi[:, 2 * H:3 * H] + r * gh[:, 2 * H:3 * H])
        h2n = (1.0 - z) * n + z * h2
        out_ref[pl.ds(i * B, B), 0:H] = h2n[0:B].astype(jnp.bfloat16)
        out_ref[pl.ds((L - 3 - i) * B, B), H:2 * H] = \
            h2n[B:2 * B].astype(jnp.bfloat16)
        return h2n

    h0 = jnp.zeros((2 * B, H), dtype=jnp.float32)
    jax.lax.fori_loop(0, L - 2, body, h0)


def _dec_kernel(x_ref, w_ref, b_ref, out_ref, s_ref, lse_ref):
    p = pl.program_id(0)
    j = pl.program_id(1)

    logits = jax.lax.dot_general(
        x_ref[...], w_ref[...].astype(jnp.bfloat16), (((1,), (1,)), ((), ())),
        preferred_element_type=jnp.float32) + b_ref[...]

    @pl.when(p == 0)
    def _():
        @pl.when(j == 0)
        def _():
            s_ref[...] = jnp.zeros((NROWS, 1), dtype=jnp.float32)

        def masked():
            col = j * VB + jax.lax.broadcasted_iota(jnp.int32, (NROWS, VB), 1)
            return jnp.where(col < V, logits, NEG_INF)

        l = jax.lax.cond(j == NVB - 1, masked, lambda: logits)
        s_ref[...] += jnp.sum(jnp.exp(l), axis=1, keepdims=True)

        @pl.when(j == NVB - 1)
        def _():
            lse_ref[...] = jnp.log(s_ref[...])

    @pl.when(p == 1)
    def _():
        out_ref[...] = logits - lse_ref[...]


def kernel(seq, seq_length, emb, w_ih_f, w_hh_f, b_ih_f, b_hh_f,
           w_ih_b, w_hh_b, b_ih_b, b_hh_b, dec_W, dec_b):
    x = jnp.take(emb, seq.reshape(-1), axis=0)
    whhT_cat = jnp.concatenate([w_hh_f.T, w_hh_b.T], axis=1)
    bhh_cat = jnp.concatenate([b_hh_f, b_hh_b]).reshape(1, 6 * H)

    feats = pl.pallas_call(
        _gru_kernel,
        out_shape=jax.ShapeDtypeStruct((NROWS, 2 * H), jnp.bfloat16),
        scratch_shapes=[
            pltpu.VMEM((L * B, 3 * H), jnp.float32),
            pltpu.VMEM((L * B, 3 * H), jnp.float32),
        ],
    )(x, w_ih_f.T, w_ih_b.T, whhT_cat, b_ih_f.reshape(1, -1),
      b_ih_b.reshape(1, -1), bhh_cat)

    logp = pl.pallas_call(
        _dec_kernel,
        grid=(2, NVB),
        in_specs=[
            pl.BlockSpec((NROWS, 2 * H), lambda p, j: (0, 0)),
            pl.BlockSpec((VB, 2 * H), lambda p, j: (j, 0)),
            pl.BlockSpec((1, VB), lambda p, j: (0, j)),
        ],
        out_specs=pl.BlockSpec((NROWS, VB), lambda p, j: (0, p * j)),
        out_shape=jax.ShapeDtypeStruct((NROWS, V), jnp.float32),
        scratch_shapes=[
            pltpu.VMEM((NROWS, 1), jnp.float32),
            pltpu.VMEM((NROWS, 1), jnp.float32),
        ],
    )(feats, dec_W, dec_b.reshape(1, V))

    return logp.reshape(L - 2, B, V)
